# baseline (speedup 1.0000x reference)
import functools

import jax
import jax.numpy as jnp
from jax import lax
from jax.experimental import pallas as pl
from jax.experimental.pallas import tpu as pltpu
from jax.experimental.pallas import tpu_sc as plsc

_NSUB = 16
_K = 80


@functools.lru_cache(maxsize=None)
def _seg_sum_program(n, W, with_cnt, CH, NT):
    CH2 = CH // 2
    NPT = (n // _NSUB) // 8 * 8
    REM = n - NPT * _NSUB

    mesh = plsc.VectorSubcoreMesh(core_axis_name="c", subcore_axis_name="s")
    f32 = jnp.float32
    out_type = [jax.ShapeDtypeStruct((n, W), f32)] * (2 * NT)
    scratch = [
        pltpu.VMEM((CH, _K), jnp.int32),
        pltpu.VMEM((CH, _K), jnp.int32),
        pltpu.VMEM((4, _K, W), f32),
        pltpu.VMEM_SHARED((n, W), f32),
        pltpu.SemaphoreType.DMA,
        pltpu.SemaphoreType.DMA,
        pltpu.SemaphoreType.DMA,
        pltpu.SemaphoreType.DMA,
    ]
    if with_cnt:
        out_type += [jax.ShapeDtypeStruct((n, 8), f32),
                     jax.ShapeDtypeStruct((n, 8), f32)]
        scratch += [
            pltpu.VMEM((_K, 8), f32),
            pltpu.VMEM_SHARED((n, 8), f32),
        ]

    @functools.partial(
        pl.kernel, mesh=mesh, out_type=tuple(out_type),
        scratch_types=tuple(scratch),
        compiler_params=pltpu.CompilerParams(use_tc_tiling_on_sc=False))
    def krn(*refs):
        ntab = 2 * NT
        tabs = refs[:ntab]
        srcr_h, dstr_h, zW_h = refs[ntab:ntab + 3]
        p = ntab + 3
        if with_cnt:
            ones_h, z16_h = refs[p:p + 2]
            p += 2
        outs = refs[p:p + ntab]
        p += ntab
        if with_cnt:
            cntA, cntB = refs[p:p + 2]
            p += 2
        idxs, idxd, rows, acc = refs[p:p + 4]
        sems = refs[p + 4:p + 8]
        if with_cnt:
            onesv, cacc = refs[p + 8:p + 10]
        c = lax.axis_index("c")
        s = lax.axis_index("s")
        base = s * NPT

        def striped(src, dst):
            pltpu.sync_copy(src.at[pl.ds(base, NPT)],
                            dst.at[pl.ds(base, NPT)])
            if REM:
                @pl.when(s == 0)
                def _():
                    pltpu.sync_copy(src.at[pl.ds(NPT * _NSUB, REM)],
                                    dst.at[pl.ds(NPT * _NSUB, REM)])

        pltpu.sync_copy(srcr_h.at[s], idxs)
        pltpu.sync_copy(dstr_h.at[s], idxd)
        if with_cnt:
            pltpu.sync_copy(ones_h, onesv)

        def run(tab_h):
            def start(j, b):
                pltpu.async_copy(tab_h.at[idxs.at[j]], rows.at[b], sems[b])

            def drain(j, b):
                pltpu.make_async_copy(tab_h.at[idxs.at[j]], rows.at[b],
                                      sems[b]).wait()

            def scat(j, b):
                pltpu.sync_copy(rows.at[b], acc.at[idxd.at[j]], add=True)

            for b in range(3):
                start(b, b)
            MQ = (CH - 3) // 4

            def body(q, carry):
                j0 = 4 * q
                for b in range(4):
                    j = j0 + b
                    drain(j, b)
                    start(j + 3, (b + 3) % 4)
                    scat(j, b)
                return carry
            lax.fori_loop(0, MQ, body, 0)
            for j in range(4 * MQ, CH):
                b = j % 4
                drain(j, b)
                if j + 3 < CH:
                    start(j + 3, (b + 3) % 4)
                scat(j, b)

        for t in range(NT):
            striped(zW_h, acc)
            if with_cnt and t == 0:
                striped(z16_h, cacc)
            plsc.subcore_barrier()

            @pl.when(c == 0)
            def _(t=t):
                run(tabs[2 * t])

            @pl.when(c == 1)
            def _(t=t):
                run(tabs[2 * t + 1])

            if with_cnt and t == 0:
                def cbody(j, carry):
                    pltpu.sync_copy(onesv, cacc.at[idxd.at[c * CH2 + j]],
                                    add=True)
                    return carry
                lax.fori_loop(0, CH2, cbody, 0)

            plsc.subcore_barrier()

            @pl.when(c == 0)
            def _(t=t):
                striped(acc, outs[2 * t])
                if with_cnt and t == 0:
                    striped(cacc, cntA)

            @pl.when(c == 1)
            def _(t=t):
                striped(acc, outs[2 * t + 1])
                if with_cnt and t == 0:
                    striped(cacc, cntB)

    return krn


def _seg_sum_call(tables, src_r, dst_r, with_cnt=False, ones16=None,
                  zeros16=None):
    n, W2 = tables[0].shape
    h = W2 // 2
    CH = src_r.shape[1]
    NT = len(tables)
    halves = []
    for t in tables:
        halves += [t[:, :h], t[:, h:]]
    zerosW = jnp.zeros((n, h), jnp.float32)
    krn = _seg_sum_program(n, h, with_cnt, CH, NT)
    if with_cnt:
        res = krn(*halves, src_r, dst_r, zerosW, ones16, zeros16)
    else:
        res = krn(*halves, src_r, dst_r, zerosW)
        res = list(res) if isinstance(res, (list, tuple)) else [res]
    outs = [jnp.concatenate([res[2 * t], res[2 * t + 1]], axis=1)
            for t in range(NT)]
    if with_cnt:
        return outs, res[2 * NT], res[2 * NT + 1]
    return outs


def _rowblock(body, n, B, ins, n_out, out_dims):
    f32 = jnp.float32
    in_specs = []
    for a in ins:
        if a.shape[0] == n:
            in_specs.append(pl.BlockSpec(
                (B,) + a.shape[1:],
                lambda i, nd=a.ndim: (i,) + (0,) * (nd - 1)))
        else:
            in_specs.append(pl.BlockSpec(
                a.shape, lambda i, nd=a.ndim: (0,) * nd))
    out_specs = [pl.BlockSpec((B, d), lambda i: (i, 0)) for d in out_dims]
    out_shape = [jax.ShapeDtypeStruct((n, d), f32) for d in out_dims]
    return pl.pallas_call(
        body, grid=(n // B,), in_specs=in_specs,
        out_specs=out_specs, out_shape=out_shape)(*ins)


def _softplus(v):
    return jnp.log(1.0 + jnp.exp(-jnp.abs(v))) + jnp.maximum(v, 0.0)


def _inv_cnt(cA_ref, cB_ref):
    cnt = jnp.maximum(cA_ref[...][:, 0:1] + cB_ref[...][:, 0:1], 1.0)
    return 1.0 / cnt


def _dot(a, b):
    return jnp.dot(a, b, preferred_element_type=jnp.float32)


def _tc0(x, h0, diff, phiW, phib, eWl_x, eWl_h, pW_h, pW_d, pb,
         pmW, pmb, psW, psb, *, n, B):
    def body(x_r, h_r, d_r, phiW_r, phib_r, eWlx_r, eWlh_r, pWh_r, pWd_r,
             pb_r, pmW_r, pmb_r, psW_r, psb_r,
             phiX_o, S1a_o, pm_o, ps_o):
        xx = x_r[...]
        hh = h_r[...]
        phiX = jnp.maximum(_dot(xx, phiW_r[...]) + phib_r[...], 0.0)
        phiX_o[...] = phiX
        S1a_o[...] = _dot(phiX, eWlx_r[...]) + _dot(hh, eWlh_r[...])
        px = jnp.maximum(_dot(hh, pWh_r[...]) + d_r[...] * pWd_r[...]
                         + pb_r[...], 0.0)
        pm_o[...] = _dot(px, pmW_r[...]) + pmb_r[...]
        ps_o[...] = _softplus(_dot(px, psW_r[...]) + psb_r[...])
    return _rowblock(body, n, B,
                     [x, h0, diff, phiW, phib, eWl_x, eWl_h, pW_h, pW_d, pb,
                      pmW, pmb, psW, psb],
                     4, [128, 128, 64, 64])


def _tc1(M1a, M1b, cA, cB, phiX, h0, eWr_x, eWr_h, eb, emWl, esWl, *, n, B):
    def body(M1a_r, M1b_r, cA_r, cB_r, phiX_r, h0_r, eWrx_r, eWrh_r, eb_r,
             emWl_r, esWl_r, encx_o, meanH_o, S2_o):
        inv = _inv_cnt(cA_r, cB_r)
        enc_x = jnp.maximum(
            M1a_r[...] * inv + _dot(phiX_r[...], eWrx_r[...])
            + _dot(h0_r[...], eWrh_r[...]) + eb_r[...], 0.0)
        encx_o[...] = enc_x
        meanH_o[...] = M1b_r[...] * inv
        S2_o[...] = jnp.concatenate(
            [_dot(enc_x, emWl_r[...]), _dot(enc_x, esWl_r[...])], axis=1)
    return _rowblock(body, n, B,
                     [M1a, M1b, cA, cB, phiX, h0, eWr_x, eWr_h, eb, emWl, esWl],
                     3, [128, 128, 128])


def _tc2(M2, cA, cB, enc_x, noise, emWr, emb, esWr, esb, pzW, pzb,
         *, n, B):
    def body(M2_r, cA_r, cB_r, encx_r, noise_r, emWr_r, emb_r,
             esWr_r, esb_r, pzW_r, pzb_r, em_o, es_o, z_o, phiZ_o):
        inv = _inv_cnt(cA_r, cB_r)
        enc_x = encx_r[...]
        m2 = M2_r[...] * inv
        hw = m2.shape[1] // 2
        enc_mean = m2[:, :hw] + _dot(enc_x, emWr_r[...]) + emb_r[...]
        enc_std = _softplus(m2[:, hw:] + _dot(enc_x, esWr_r[...])
                            + esb_r[...])
        z = noise_r[...] * enc_std + enc_mean
        em_o[...] = enc_mean
        es_o[...] = enc_std
        z_o[...] = z
        phiZ_o[...] = jnp.maximum(_dot(z, pzW_r[...]) + pzb_r[...], 0.0)
    return _rowblock(body, n, B,
                     [M2, cA, cB, enc_x, noise, emWr, emb, esWr, esb,
                      pzW, pzb],
                     4, [64, 64, 64, 128])


def _tc3(M3a, M3b, cA, cB, meanH, phiX, phiZ, h0,
         xzWla, xzWlb, xzWra, xzWrb, xzb, hzWl, hzWr, hzb,
         xrWla, xrWlb, xrWra, xrWrb, xrb, hrWl, hrWr, hrb,
         hhWl_a, hhWl_b, *, n, B):
    def body(M3a_r, M3b_r, cA_r, cB_r, meanH_r, phiX_r, phiZ_r, h0_r,
             xzWla_r, xzWlb_r, xzWra_r, xzWrb_r, xzb_r, hzWl_r, hzWr_r,
             hzb_r, xrWla_r, xrWlb_r, xrWra_r, xrWrb_r, xrb_r, hrWl_r,
             hrWr_r, hrb_r, hhWla_r, hhWlb_r,
             zg_o, rh_o, S4_o):
        inv = _inv_cnt(cA_r, cB_r)
        m3a = M3a_r[...] * inv
        m3b = M3b_r[...] * inv
        phiX = phiX_r[...]
        phiZ = phiZ_r[...]
        h0 = h0_r[...]
        meanH = meanH_r[...]
        zg = jax.nn.sigmoid(
            _dot(m3a, xzWla_r[...]) + _dot(m3b, xzWlb_r[...])
            + _dot(phiX, xzWra_r[...]) + _dot(phiZ, xzWrb_r[...]) + xzb_r[...]
            + _dot(meanH, hzWl_r[...]) + _dot(h0, hzWr_r[...]) + hzb_r[...])
        rg = jax.nn.sigmoid(
            _dot(m3a, xrWla_r[...]) + _dot(m3b, xrWlb_r[...])
            + _dot(phiX, xrWra_r[...]) + _dot(phiZ, xrWrb_r[...]) + xrb_r[...]
            + _dot(meanH, hrWl_r[...]) + _dot(h0, hrWr_r[...]) + hrb_r[...])
        rh = rg * h0
        zg_o[...] = zg
        rh_o[...] = rh
        S4_o[...] = jnp.concatenate(
            [_dot(rh, hhWla_r[...]), _dot(rh, hhWlb_r[...])], axis=1)
    return _rowblock(body, n, B,
                     [M3a, M3b, cA, cB, meanH, phiX, phiZ, h0,
                      xzWla, xzWlb, xzWra, xzWrb, xzb, hzWl, hzWr, hzb,
                      xrWla, xrWlb, xrWra, xrWrb, xrb, hrWl, hrWr, hrb,
                      hhWl_a, hhWl_b],
                     3, [128, 128, 128])


def _tc4(M4, M3a, M3b, cA, cB, phiX, phiZ, rh, zg, h0,
         xhWla, xhWlb, xhWra, xhWrb, xhb, hhWr, hhb, *, n, B):
    def body(M4_r, M3a_r, M3b_r, cA_r, cB_r, phiX_r, phiZ_r, rh_r,
             zg_r, h0_r, xhWla_r, xhWlb_r, xhWra_r, xhWrb_r, xhb_r, hhWr_r,
             hhb_r, out_o):
        inv = _inv_cnt(cA_r, cB_r)
        m4 = M4_r[...] * inv
        ht = jnp.tanh(
            _dot(M3a_r[...] * inv, xhWla_r[...])
            + _dot(M3b_r[...] * inv, xhWlb_r[...])
            + _dot(phiX_r[...], xhWra_r[...]) + _dot(phiZ_r[...], xhWrb_r[...])
            + xhb_r[...] + m4 + _dot(rh_r[...], hhWr_r[...]) + hhb_r[...])
        zg = zg_r[...]
        out_o[...] = zg * h0_r[...] + (1.0 - zg) * ht
    return _rowblock(body, n, B,
                     [M4, M3a, M3b, cA, cB, phiX, phiZ, rh, zg, h0,
                      xhWla, xhWlb, xhWra, xhWrb, xhb, hhWr, hhb],
                     1, [128])


@jax.jit
def kernel(x, h, diff, noise, edge_index, phi_x_W, phi_x_b, phi_z_W, phi_z_b,
           prior_W, prior_b, priorm_W, priorm_b, priors_W, priors_b,
           enc_Wl, enc_Wr, enc_b,
           encm_Wl, encm_Wr, encm_b,
           encs_Wl, encs_Wr, encs_b,
           xz_Wl, xz_Wr, xz_b, hz_Wl, hz_Wr, hz_b,
           xr_Wl, xr_Wr, xr_b, hr_Wl, hr_Wr, hr_b,
           xh_Wl, xh_Wr, xh_b, hh_Wl, hh_Wr, hh_b):
    n = x.shape[0]
    e = edge_index.shape[1]
    hd = h.shape[2]
    B = 1000
    CH = e // (_NSUB * _K)
    h0 = h[0]

    src_r = edge_index[0].reshape(_NSUB, CH, _K)
    dst_r = edge_index[1].reshape(_NSUB, CH, _K)
    ones16 = jnp.ones((_K, 8), jnp.float32)
    zeros16 = jnp.zeros((n, 8), jnp.float32)

    r2 = lambda v: v.reshape(1, -1)
    phiX, S1a, prior_mean, prior_std = _tc0(
        x, h0, diff, phi_x_W, r2(phi_x_b), enc_Wl[:hd], enc_Wl[hd:],
        prior_W[:hd], prior_W[hd:hd + 1], r2(prior_b),
        priorm_W, r2(priorm_b), priors_W, r2(priors_b), n=n, B=B)

    (M1a, M1b, M3a), cntA, cntB = _seg_sum_call(
        [S1a, h0, phiX], src_r, dst_r, True, ones16, zeros16)

    enc_x, meanH, S2 = _tc1(
        M1a, M1b, cntA, cntB, phiX, h0, enc_Wr[:hd], enc_Wr[hd:], r2(enc_b),
        encm_Wl, encs_Wl, n=n, B=B)

    (M2,) = _seg_sum_call([S2], src_r, dst_r)

    enc_mean, enc_std, z, phiZ = _tc2(
        M2, cntA, cntB, enc_x, noise, encm_Wr, r2(encm_b), encs_Wr,
        r2(encs_b), phi_z_W, r2(phi_z_b), n=n, B=B)

    (M3b,) = _seg_sum_call([phiZ], src_r, dst_r)

    zg, rh, S4 = _tc3(
        M3a, M3b, cntA, cntB, meanH, phiX, phiZ, h0,
        xz_Wl[:hd], xz_Wl[hd:], xz_Wr[:hd], xz_Wr[hd:], r2(xz_b),
        hz_Wl, hz_Wr, r2(hz_b),
        xr_Wl[:hd], xr_Wl[hd:], xr_Wr[:hd], xr_Wr[hd:], r2(xr_b),
        hr_Wl, hr_Wr, r2(hr_b),
        hh_Wl[:, :64], hh_Wl[:, 64:], n=n, B=B)

    (M4,) = _seg_sum_call([S4], src_r, dst_r)

    (out,) = _tc4(
        M4, M3a, M3b, cntA, cntB, phiX, phiZ, rh, zg, h0,
        xh_Wl[:hd], xh_Wl[hd:], xh_Wr[:hd], xh_Wr[hd:], r2(xh_b),
        hh_Wr, r2(hh_b), n=n, B=B)

    return (prior_mean, prior_std, enc_mean, enc_std, z, out[None])


# speedup vs baseline: 1.0908x; 1.0908x over previous
import functools

import jax
import jax.numpy as jnp
from jax import lax
from jax.experimental import pallas as pl
from jax.experimental.pallas import tpu as pltpu
from jax.experimental.pallas import tpu_sc as plsc

_NSUB = 16
_K = 80


@functools.lru_cache(maxsize=None)
def _seg_sum_program(n, W, with_cnt, CH):
    CH2 = CH // 2
    NPT = (n // _NSUB) // 8 * 8
    REM = n - NPT * _NSUB

    mesh = plsc.VectorSubcoreMesh(core_axis_name="c", subcore_axis_name="s")
    f32 = jnp.float32
    out_type = [jax.ShapeDtypeStruct((n, W), f32),
                jax.ShapeDtypeStruct((n, W), f32)]
    scratch = [
        pltpu.VMEM((CH, _K), jnp.int32),
        pltpu.VMEM((CH, _K), jnp.int32),
        pltpu.VMEM((4, _K, W), f32),
        pltpu.VMEM_SHARED((n, W), f32),
        pltpu.SemaphoreType.DMA,
        pltpu.SemaphoreType.DMA,
        pltpu.SemaphoreType.DMA,
        pltpu.SemaphoreType.DMA,
    ]
    if with_cnt:
        out_type += [jax.ShapeDtypeStruct((n, 8), f32),
                     jax.ShapeDtypeStruct((n, 8), f32)]
        scratch += [
            pltpu.VMEM((_K, 8), f32),
            pltpu.VMEM_SHARED((n, 8), f32),
        ]

    @functools.partial(
        pl.kernel, mesh=mesh, out_type=tuple(out_type),
        scratch_types=tuple(scratch),
        compiler_params=pltpu.CompilerParams(use_tc_tiling_on_sc=False))
    def krn(*refs):
        if with_cnt:
            (tabA_h, tabB_h, srcr_h, dstr_h, zW_h, ones_h, z16_h,
             outA, outB, cntA, cntB,
             idxs, idxd, rows, acc, sem0, sem1, sem2, sem3,
             onesv, cacc) = refs
        else:
            (tabA_h, tabB_h, srcr_h, dstr_h, zW_h,
             outA, outB,
             idxs, idxd, rows, acc, sem0, sem1, sem2, sem3) = refs
        sems = (sem0, sem1, sem2, sem3)
        c = lax.axis_index("c")
        s = lax.axis_index("s")
        base = s * NPT

        def striped(src, dst):
            pltpu.sync_copy(src.at[pl.ds(base, NPT)],
                            dst.at[pl.ds(base, NPT)])
            if REM:
                @pl.when(s == 0)
                def _():
                    pltpu.sync_copy(src.at[pl.ds(NPT * _NSUB, REM)],
                                    dst.at[pl.ds(NPT * _NSUB, REM)])

        pltpu.sync_copy(srcr_h.at[s], idxs)
        pltpu.sync_copy(dstr_h.at[s], idxd)
        striped(zW_h, acc)
        if with_cnt:
            striped(z16_h, cacc)
            pltpu.sync_copy(ones_h, onesv)
        plsc.subcore_barrier()

        def run(tab_h):
            def start(j, b):
                pltpu.async_copy(tab_h.at[idxs.at[j]], rows.at[b], sems[b])

            def drain(j, b):
                pltpu.make_async_copy(tab_h.at[idxs.at[j]], rows.at[b],
                                      sems[b]).wait()

            def scat(j, b):
                pltpu.sync_copy(rows.at[b], acc.at[idxd.at[j]], add=True)

            for b in range(3):
                start(b, b)
            MQ = (CH - 3) // 4

            def body(q, carry):
                j0 = 4 * q
                for b in range(4):
                    j = j0 + b
                    drain(j, b)
                    start(j + 3, (b + 3) % 4)
                    scat(j, b)
                return carry
            lax.fori_loop(0, MQ, body, 0)
            for j in range(4 * MQ, CH):
                b = j % 4
                drain(j, b)
                if j + 3 < CH:
                    start(j + 3, (b + 3) % 4)
                scat(j, b)

        @pl.when(c == 0)
        def _():
            run(tabA_h)

        @pl.when(c == 1)
        def _():
            run(tabB_h)

        if with_cnt:
            def cbody(j, carry):
                pltpu.sync_copy(onesv, cacc.at[idxd.at[c * CH2 + j]],
                                add=True)
                return carry
            lax.fori_loop(0, CH2, cbody, 0)

        plsc.subcore_barrier()

        @pl.when(c == 0)
        def _():
            striped(acc, outA)
            if with_cnt:
                striped(cacc, cntA)

        @pl.when(c == 1)
        def _():
            striped(acc, outB)
            if with_cnt:
                striped(cacc, cntB)

    return krn


def _seg_sum_pass(tabA, tabB, src_r, dst_r, with_cnt, ones16=None,
                  zeros16=None):
    n, W = tabA.shape
    CH = src_r.shape[1]
    f32 = jnp.float32
    zerosW = jnp.zeros((n, W), f32)
    krn = _seg_sum_program(n, W, with_cnt, CH)
    if with_cnt:
        return krn(tabA, tabB, src_r, dst_r, zerosW, ones16, zeros16)
    return krn(tabA, tabB, src_r, dst_r, zerosW)


def _seg_sum_wide(tab, src_r, dst_r, with_cnt=False, ones16=None,
                  zeros16=None):
    n, W = tab.shape
    h = W // 2
    if with_cnt:
        a0, a1, cA, cB = _seg_sum_pass(tab[:, :h], tab[:, h:], src_r, dst_r,
                                       True, ones16, zeros16)
        return jnp.concatenate([a0, a1], axis=1), cA, cB
    a0, a1 = _seg_sum_pass(tab[:, :h], tab[:, h:], src_r, dst_r, False)
    return jnp.concatenate([a0, a1], axis=1)


def _rowblock(body, n, B, ins, n_out, out_dims):
    f32 = jnp.float32
    in_specs = []
    for a in ins:
        if a.shape[0] == n:
            in_specs.append(pl.BlockSpec(
                (B,) + a.shape[1:],
                lambda i, nd=a.ndim: (i,) + (0,) * (nd - 1)))
        else:
            in_specs.append(pl.BlockSpec(
                a.shape, lambda i, nd=a.ndim: (0,) * nd))
    out_specs = [pl.BlockSpec((B, d), lambda i: (i, 0)) for d in out_dims]
    out_shape = [jax.ShapeDtypeStruct((n, d), f32) for d in out_dims]
    return pl.pallas_call(
        body, grid=(n // B,), in_specs=in_specs,
        out_specs=out_specs, out_shape=out_shape)(*ins)


def _softplus(v):
    return jnp.log(1.0 + jnp.exp(-jnp.abs(v))) + jnp.maximum(v, 0.0)


def _inv_cnt(cA_ref, cB_ref):
    cnt = jnp.maximum(cA_ref[...][:, 0:1] + cB_ref[...][:, 0:1], 1.0)
    return 1.0 / cnt


def _dot(a, b):
    return jnp.dot(a, b, preferred_element_type=jnp.float32)


def _tc0(x, h0, diff, phiW, phib, eWl_x, eWl_h, pW_h, pW_d, pb,
         pmW, pmb, psW, psb, *, n, B):
    def body(x_r, h_r, d_r, phiW_r, phib_r, eWlx_r, eWlh_r, pWh_r, pWd_r,
             pb_r, pmW_r, pmb_r, psW_r, psb_r,
             phiX_o, S1a_o, pm_o, ps_o):
        xx = x_r[...]
        hh = h_r[...]
        phiX = jnp.maximum(_dot(xx, phiW_r[...]) + phib_r[...], 0.0)
        phiX_o[...] = phiX
        S1a_o[...] = _dot(phiX, eWlx_r[...]) + _dot(hh, eWlh_r[...])
        px = jnp.maximum(_dot(hh, pWh_r[...]) + d_r[...] * pWd_r[...]
                         + pb_r[...], 0.0)
        pm_o[...] = _dot(px, pmW_r[...]) + pmb_r[...]
        ps_o[...] = _softplus(_dot(px, psW_r[...]) + psb_r[...])
    return _rowblock(body, n, B,
                     [x, h0, diff, phiW, phib, eWl_x, eWl_h, pW_h, pW_d, pb,
                      pmW, pmb, psW, psb],
                     4, [128, 128, 64, 64])


def _tc1(M1a, M1b, cA, cB, phiX, h0, eWr_x, eWr_h, eb, emWl, esWl, *, n, B):
    def body(M1a_r, M1b_r, cA_r, cB_r, phiX_r, h0_r, eWrx_r, eWrh_r, eb_r,
             emWl_r, esWl_r, encx_o, meanH_o, S2a_o, S2b_o):
        inv = _inv_cnt(cA_r, cB_r)
        enc_x = jnp.maximum(
            M1a_r[...] * inv + _dot(phiX_r[...], eWrx_r[...])
            + _dot(h0_r[...], eWrh_r[...]) + eb_r[...], 0.0)
        encx_o[...] = enc_x
        meanH_o[...] = M1b_r[...] * inv
        S2a_o[...] = _dot(enc_x, emWl_r[...])
        S2b_o[...] = _dot(enc_x, esWl_r[...])
    return _rowblock(body, n, B,
                     [M1a, M1b, cA, cB, phiX, h0, eWr_x, eWr_h, eb, emWl, esWl],
                     4, [128, 128, 64, 64])


def _tc2(M2a, M2b, cA, cB, enc_x, noise, emWr, emb, esWr, esb, pzW, pzb,
         *, n, B):
    def body(M2a_r, M2b_r, cA_r, cB_r, encx_r, noise_r, emWr_r, emb_r,
             esWr_r, esb_r, pzW_r, pzb_r, em_o, es_o, z_o, phiZ_o):
        inv = _inv_cnt(cA_r, cB_r)
        enc_x = encx_r[...]
        enc_mean = M2a_r[...] * inv + _dot(enc_x, emWr_r[...]) + emb_r[...]
        enc_std = _softplus(M2b_r[...] * inv + _dot(enc_x, esWr_r[...])
                            + esb_r[...])
        z = noise_r[...] * enc_std + enc_mean
        em_o[...] = enc_mean
        es_o[...] = enc_std
        z_o[...] = z
        phiZ_o[...] = jnp.maximum(_dot(z, pzW_r[...]) + pzb_r[...], 0.0)
    return _rowblock(body, n, B,
                     [M2a, M2b, cA, cB, enc_x, noise, emWr, emb, esWr, esb,
                      pzW, pzb],
                     4, [64, 64, 64, 128])


def _tc3(M3a, M3b, cA, cB, meanH, phiX, phiZ, h0,
         xzWla, xzWlb, xzWra, xzWrb, xzb, hzWl, hzWr, hzb,
         xrWla, xrWlb, xrWra, xrWrb, xrb, hrWl, hrWr, hrb,
         hhWl_a, hhWl_b, *, n, B):
    def body(M3a_r, M3b_r, cA_r, cB_r, meanH_r, phiX_r, phiZ_r, h0_r,
             xzWla_r, xzWlb_r, xzWra_r, xzWrb_r, xzb_r, hzWl_r, hzWr_r,
             hzb_r, xrWla_r, xrWlb_r, xrWra_r, xrWrb_r, xrb_r, hrWl_r,
             hrWr_r, hrb_r, hhWla_r, hhWlb_r,
             zg_o, rh_o, S4a_o, S4b_o):
        inv = _inv_cnt(cA_r, cB_r)
        m3a = M3a_r[...] * inv
        m3b = M3b_r[...] * inv
        phiX = phiX_r[...]
        phiZ = phiZ_r[...]
        h0 = h0_r[...]
        meanH = meanH_r[...]
        zg = jax.nn.sigmoid(
            _dot(m3a, xzWla_r[...]) + _dot(m3b, xzWlb_r[...])
            + _dot(phiX, xzWra_r[...]) + _dot(phiZ, xzWrb_r[...]) + xzb_r[...]
            + _dot(meanH, hzWl_r[...]) + _dot(h0, hzWr_r[...]) + hzb_r[...])
        rg = jax.nn.sigmoid(
            _dot(m3a, xrWla_r[...]) + _dot(m3b, xrWlb_r[...])
            + _dot(phiX, xrWra_r[...]) + _dot(phiZ, xrWrb_r[...]) + xrb_r[...]
            + _dot(meanH, hrWl_r[...]) + _dot(h0, hrWr_r[...]) + hrb_r[...])
        rh = rg * h0
        zg_o[...] = zg
        rh_o[...] = rh
        S4a_o[...] = _dot(rh, hhWla_r[...])
        S4b_o[...] = _dot(rh, hhWlb_r[...])
    return _rowblock(body, n, B,
                     [M3a, M3b, cA, cB, meanH, phiX, phiZ, h0,
                      xzWla, xzWlb, xzWra, xzWrb, xzb, hzWl, hzWr, hzb,
                      xrWla, xrWlb, xrWra, xrWrb, xrb, hrWl, hrWr, hrb,
                      hhWl_a, hhWl_b],
                     4, [128, 128, 64, 64])


def _tc4(M4a, M4b, M3a, M3b, cA, cB, phiX, phiZ, rh, zg, h0,
         xhWla, xhWlb, xhWra, xhWrb, xhb, hhWr, hhb, *, n, B):
    def body(M4a_r, M4b_r, M3a_r, M3b_r, cA_r, cB_r, phiX_r, phiZ_r, rh_r,
             zg_r, h0_r, xhWla_r, xhWlb_r, xhWra_r, xhWrb_r, xhb_r, hhWr_r,
             hhb_r, out_o):
        inv = _inv_cnt(cA_r, cB_r)
        m4 = jnp.concatenate([M4a_r[...] * inv, M4b_r[...] * inv], axis=1)
        ht = jnp.tanh(
            _dot(M3a_r[...] * inv, xhWla_r[...])
            + _dot(M3b_r[...] * inv, xhWlb_r[...])
            + _dot(phiX_r[...], xhWra_r[...]) + _dot(phiZ_r[...], xhWrb_r[...])
            + xhb_r[...] + m4 + _dot(rh_r[...], hhWr_r[...]) + hhb_r[...])
        zg = zg_r[...]
        out_o[...] = zg * h0_r[...] + (1.0 - zg) * ht
    return _rowblock(body, n, B,
                     [M4a, M4b, M3a, M3b, cA, cB, phiX, phiZ, rh, zg, h0,
                      xhWla, xhWlb, xhWra, xhWrb, xhb, hhWr, hhb],
                     1, [128])


@jax.jit
def kernel(x, h, diff, noise, edge_index, phi_x_W, phi_x_b, phi_z_W, phi_z_b,
           prior_W, prior_b, priorm_W, priorm_b, priors_W, priors_b,
           enc_Wl, enc_Wr, enc_b,
           encm_Wl, encm_Wr, encm_b,
           encs_Wl, encs_Wr, encs_b,
           xz_Wl, xz_Wr, xz_b, hz_Wl, hz_Wr, hz_b,
           xr_Wl, xr_Wr, xr_b, hr_Wl, hr_Wr, hr_b,
           xh_Wl, xh_Wr, xh_b, hh_Wl, hh_Wr, hh_b):
    n = x.shape[0]
    e = edge_index.shape[1]
    hd = h.shape[2]
    B = 2000
    CH = e // (_NSUB * _K)
    h0 = h[0]

    src_r = edge_index[0].reshape(_NSUB, CH, _K)
    dst_r = edge_index[1].reshape(_NSUB, CH, _K)
    ones16 = jnp.ones((_K, 8), jnp.float32)
    zeros16 = jnp.zeros((n, 8), jnp.float32)

    r2 = lambda v: v.reshape(1, -1)
    M1b = _seg_sum_wide(h0, src_r, dst_r)

    phiX, S1a, prior_mean, prior_std = _tc0(
        x, h0, diff, phi_x_W, r2(phi_x_b), enc_Wl[:hd], enc_Wl[hd:],
        prior_W[:hd], prior_W[hd:hd + 1], r2(prior_b),
        priorm_W, r2(priorm_b), priors_W, r2(priors_b), n=n, B=B)

    M1a, cntA, cntB = _seg_sum_wide(S1a, src_r, dst_r, True, ones16, zeros16)

    M3a = _seg_sum_wide(phiX, src_r, dst_r)

    enc_x, meanH, S2a, S2b = _tc1(
        M1a, M1b, cntA, cntB, phiX, h0, enc_Wr[:hd], enc_Wr[hd:], r2(enc_b),
        encm_Wl, encs_Wl, n=n, B=B)

    M2a, M2b = _seg_sum_pass(S2a, S2b, src_r, dst_r, False)

    enc_mean, enc_std, z, phiZ = _tc2(
        M2a, M2b, cntA, cntB, enc_x, noise, encm_Wr, r2(encm_b), encs_Wr,
        r2(encs_b), phi_z_W, r2(phi_z_b), n=n, B=B)

    M3b = _seg_sum_wide(phiZ, src_r, dst_r)

    zg, rh, S4a, S4b = _tc3(
        M3a, M3b, cntA, cntB, meanH, phiX, phiZ, h0,
        xz_Wl[:hd], xz_Wl[hd:], xz_Wr[:hd], xz_Wr[hd:], r2(xz_b),
        hz_Wl, hz_Wr, r2(hz_b),
        xr_Wl[:hd], xr_Wl[hd:], xr_Wr[:hd], xr_Wr[hd:], r2(xr_b),
        hr_Wl, hr_Wr, r2(hr_b),
        hh_Wl[:, :64], hh_Wl[:, 64:], n=n, B=B)

    M4a, M4b = _seg_sum_pass(S4a, S4b, src_r, dst_r, False)

    (out,) = _tc4(
        M4a, M4b, M3a, M3b, cntA, cntB, phiX, phiZ, rh, zg, h0,
        xh_Wl[:hd], xh_Wl[hd:], xh_Wr[:hd], xh_Wr[hd:], r2(xh_b),
        hh_Wr, r2(hh_b), n=n, B=B)

    return (prior_mean, prior_std, enc_mean, enc_std, z, out[None])
